# baseline (speedup 1.0000x reference)
import functools

import jax
import jax.numpy as jnp
from jax import lax
from jax.experimental import pallas as pl
from jax.experimental.pallas import tpu as pltpu
from jax.experimental.pallas import tpu_sc as plsc

N_NODES = 10000
DIM = 128
N_EDGES = 320000

NC = 2
NS = 16
NW = NC * NS

EB = 128
BLK_PER_TILE = 80
NBLK_PAD = NW * BLK_PER_TILE
N_EDGES_PAD = NBLK_PAD * EB
AGG_ROWS = N_NODES + 8
ROWS_PER_TILE = 624
ROWS_TAIL = N_NODES - NS * ROWS_PER_TILE
NBUF = 2
HALF = BLK_PER_TILE // 2


def _segsum_body(x_hbm, src_hbm, dst_hbm, zeros_hbm, out_hbm,
                 idx_s, idx_d, rows, agg_sh, sems):
    c = lax.axis_index("c")
    s = lax.axis_index("s")
    wid = c * NS + s
    iboff = pl.multiple_of(wid * BLK_PER_TILE, 8)

    base = pl.multiple_of(s * ROWS_PER_TILE, 8)
    pltpu.sync_copy(zeros_hbm.at[pl.ds(base, ROWS_PER_TILE)],
                    agg_sh.at[pl.ds(base, ROWS_PER_TILE)])

    @pl.when(s == NS - 1)
    def _():
        pltpu.sync_copy(zeros_hbm.at[pl.ds(NS * ROWS_PER_TILE,
                                           ROWS_TAIL + 8)],
                        agg_sh.at[pl.ds(NS * ROWS_PER_TILE, ROWS_TAIL + 8)])

    plsc.subcore_barrier()

    def gather(b, blk):
        return pltpu.async_copy(x_hbm.at[idx_s.at[blk]], rows.at[b], sems[b])

    for p in range(2):
        pltpu.sync_copy(src_hbm.at[pl.ds(iboff + p * HALF, HALF)], idx_s)
        pltpu.sync_copy(dst_hbm.at[pl.ds(iboff + p * HALF, HALF)], idx_d)

        for b in range(NBUF):
            gather(b, b)

        def body(j, carry):
            for b in range(NBUF):
                blk = j + b
                pltpu.make_async_copy(x_hbm.at[idx_s.at[blk]], rows.at[b],
                                      sems[b]).wait()
                pltpu.sync_copy(rows.at[b], agg_sh.at[idx_d.at[blk]],
                                add=True)

                @pl.when(blk + NBUF < HALF)
                def _():
                    gather(b, blk + NBUF)

            return carry

        lax.fori_loop(0, HALF // NBUF, lambda j, cr: body(j * NBUF, cr), 0)

    plsc.subcore_barrier()
    pltpu.sync_copy(agg_sh.at[pl.ds(base, ROWS_PER_TILE)],
                    out_hbm.at[c, pl.ds(base, ROWS_PER_TILE)])

    @pl.when(s == NS - 1)
    def _():
        pltpu.sync_copy(agg_sh.at[pl.ds(NS * ROWS_PER_TILE, ROWS_TAIL)],
                        out_hbm.at[c, pl.ds(NS * ROWS_PER_TILE, ROWS_TAIL)])


@jax.jit
def _sc_segsum(x, src, dst, zeros):
    mesh = plsc.VectorSubcoreMesh(core_axis_name="c", subcore_axis_name="s")
    return pl.kernel(
        _segsum_body,
        out_type=jax.ShapeDtypeStruct((NC, N_NODES, DIM), jnp.float32),
        mesh=mesh,
        scratch_types=[
            pltpu.VMEM((HALF, EB), jnp.int32),
            pltpu.VMEM((HALF, EB), jnp.int32),
            pltpu.VMEM((NBUF, EB, DIM), jnp.float32),
            pltpu.VMEM_SHARED((AGG_ROWS, DIM), jnp.float32),
            [pltpu.SemaphoreType.DMA] * NBUF,
        ],
    )(x, src, dst, zeros)


ROW_BLK = 1000


def _mlp_body(h_ref, agg_ref, wa_ref, wb_ref, wc_ref, l_ref,
              hout_ref, head_ref, acc_ref):
    i = pl.program_id(0)
    z = h_ref[...] + agg_ref[0] + agg_ref[1]
    z = jnp.maximum(jnp.dot(z, wa_ref[...], preferred_element_type=jnp.float32), 0.0)
    z = jnp.maximum(jnp.dot(z, wb_ref[...], preferred_element_type=jnp.float32), 0.0)
    z = jnp.maximum(jnp.dot(z, wc_ref[...], preferred_element_type=jnp.float32), 0.0)
    hout_ref[...] = z

    @pl.when(i == 0)
    def _():
        acc_ref[...] = jnp.zeros_like(acc_ref)

    acc_ref[...] += jnp.sum(z, axis=0, keepdims=True)

    @pl.when(i == pl.num_programs(0) - 1)
    def _():
        head_ref[...] = jnp.dot(acc_ref[...] / N_NODES, l_ref[...],
                                preferred_element_type=jnp.float32)


def _final_body(h_ref, agg_ref, wa_ref, wb_ref, wc_ref, l_ref, prev_ref,
                out_ref, acc_ref):
    i = pl.program_id(0)
    z = h_ref[...] + agg_ref[0] + agg_ref[1]
    z = jnp.maximum(jnp.dot(z, wa_ref[...], preferred_element_type=jnp.float32), 0.0)
    z = jnp.maximum(jnp.dot(z, wb_ref[...], preferred_element_type=jnp.float32), 0.0)
    z = jnp.maximum(jnp.dot(z, wc_ref[...], preferred_element_type=jnp.float32), 0.0)

    @pl.when(i == 0)
    def _():
        acc_ref[...] = jnp.zeros_like(acc_ref)

    acc_ref[...] += jnp.sum(z, axis=0, keepdims=True)

    @pl.when(i == pl.num_programs(0) - 1)
    def _():
        head = jnp.dot(acc_ref[...] / N_NODES, l_ref[...],
                       preferred_element_type=jnp.float32)
        total = head + jnp.sum(prev_ref[...], axis=0, keepdims=True)
        out_ref[...] = jax.nn.sigmoid(total)


def _tc_mlp(h, agg, wa, wb, wc, l):
    grid = N_NODES // ROW_BLK
    return pl.pallas_call(
        _mlp_body,
        grid=(grid,),
        in_specs=[
            pl.BlockSpec((ROW_BLK, DIM), lambda i: (i, 0)),
            pl.BlockSpec((NC, ROW_BLK, DIM), lambda i: (0, i, 0)),
            pl.BlockSpec((DIM, DIM), lambda i: (0, 0)),
            pl.BlockSpec((DIM, DIM), lambda i: (0, 0)),
            pl.BlockSpec((DIM, DIM), lambda i: (0, 0)),
            pl.BlockSpec((DIM, 1), lambda i: (0, 0)),
        ],
        out_specs=[
            pl.BlockSpec((ROW_BLK, DIM), lambda i: (i, 0)),
            pl.BlockSpec((1, 1), lambda i: (0, 0)),
        ],
        out_shape=[
            jax.ShapeDtypeStruct((N_NODES, DIM), jnp.float32),
            jax.ShapeDtypeStruct((1, 1), jnp.float32),
        ],
        scratch_shapes=[pltpu.VMEM((1, DIM), jnp.float32)],
    )(h, agg, wa, wb, wc, l)


def _tc_final(h, agg, wa, wb, wc, l, prev):
    grid = N_NODES // ROW_BLK
    return pl.pallas_call(
        _final_body,
        grid=(grid,),
        in_specs=[
            pl.BlockSpec((ROW_BLK, DIM), lambda i: (i, 0)),
            pl.BlockSpec((NC, ROW_BLK, DIM), lambda i: (0, i, 0)),
            pl.BlockSpec((DIM, DIM), lambda i: (0, 0)),
            pl.BlockSpec((DIM, DIM), lambda i: (0, 0)),
            pl.BlockSpec((DIM, DIM), lambda i: (0, 0)),
            pl.BlockSpec((DIM, 1), lambda i: (0, 0)),
            pl.BlockSpec((4, 1), lambda i: (0, 0)),
        ],
        out_specs=pl.BlockSpec((1, 1), lambda i: (0, 0)),
        out_shape=jax.ShapeDtypeStruct((1, 1), jnp.float32),
        scratch_shapes=[pltpu.VMEM((1, DIM), jnp.float32)],
    )(h, agg, wa, wb, wc, l, prev)


def kernel(x, edge_index, Wa, Wb, Wc, L):
    npad = N_EDGES_PAD - N_EDGES
    src = jnp.concatenate(
        [edge_index[0], jnp.zeros((npad,), jnp.int32)]).reshape(NBLK_PAD, EB)
    dst = jnp.concatenate(
        [edge_index[1],
         N_NODES + (jnp.arange(npad, dtype=jnp.int32) % 8)]).reshape(NBLK_PAD, EB)
    zeros = jnp.zeros((AGG_ROWS, DIM), jnp.float32)

    h = x
    heads = []
    for i in range(4):
        agg = _sc_segsum(h, src, dst, zeros)
        h, head = _tc_mlp(h, agg, Wa[i], Wb[i], Wc[i], L[i])
        heads.append(head)

    agg = _sc_segsum(h, src, dst, zeros)
    prev = jnp.concatenate(heads, axis=0)
    out = _tc_final(h, agg, Wa[4], Wb[4], Wc[4], L[4], prev)
    return out.reshape((1,))


# speedup vs baseline: 3.5765x; 3.5765x over previous
import functools

import jax
import jax.numpy as jnp
from jax import lax
from jax.experimental import pallas as pl
from jax.experimental.pallas import tpu as pltpu
from jax.experimental.pallas import tpu_sc as plsc

N_NODES = 10000
DIM = 128
N_EDGES = 320000

NC = 2
NS = 16
NW = NC * NS

EB = 125
BLK_PER_TILE = 80
NBLK = NW * BLK_PER_TILE
AGG_ROWS = N_NODES
ROWS_PER_TILE = 624
ROWS_TAIL = N_NODES - NS * ROWS_PER_TILE
NBUF = 2
HALF = BLK_PER_TILE // 2


def _segsum_body(x_hbm, src_hbm, dst_hbm, zeros_hbm, out_hbm,
                 idx_s, idx_d, rows, agg_sh, sems):
    c = lax.axis_index("c")
    s = lax.axis_index("s")
    wid = c * NS + s
    iboff = pl.multiple_of(wid * BLK_PER_TILE, 8)

    base = pl.multiple_of(s * ROWS_PER_TILE, 8)
    pltpu.sync_copy(zeros_hbm.at[pl.ds(base, ROWS_PER_TILE)],
                    agg_sh.at[pl.ds(base, ROWS_PER_TILE)])

    @pl.when(s == NS - 1)
    def _():
        pltpu.sync_copy(zeros_hbm.at[pl.ds(NS * ROWS_PER_TILE, ROWS_TAIL)],
                        agg_sh.at[pl.ds(NS * ROWS_PER_TILE, ROWS_TAIL)])

    plsc.subcore_barrier()

    def gather(b, blk):
        return pltpu.async_copy(x_hbm.at[idx_s.at[blk]], rows.at[b], sems[b])

    for p in range(2):
        pltpu.sync_copy(src_hbm.at[pl.ds(iboff + p * HALF, HALF)], idx_s)
        pltpu.sync_copy(dst_hbm.at[pl.ds(iboff + p * HALF, HALF)], idx_d)

        for b in range(NBUF):
            gather(b, b)

        def body(j, carry):
            for b in range(NBUF):
                blk = j + b
                pltpu.make_async_copy(x_hbm.at[idx_s.at[blk]], rows.at[b],
                                      sems[b]).wait()
                pltpu.sync_copy(rows.at[b], agg_sh.at[idx_d.at[blk]],
                                add=True)

                @pl.when(blk + NBUF < HALF)
                def _():
                    gather(b, blk + NBUF)

            return carry

        lax.fori_loop(0, HALF // NBUF, lambda j, cr: body(j * NBUF, cr), 0)

    plsc.subcore_barrier()
    pltpu.sync_copy(agg_sh.at[pl.ds(base, ROWS_PER_TILE)],
                    out_hbm.at[c, pl.ds(base, ROWS_PER_TILE)])

    @pl.when(s == NS - 1)
    def _():
        pltpu.sync_copy(agg_sh.at[pl.ds(NS * ROWS_PER_TILE, ROWS_TAIL)],
                        out_hbm.at[c, pl.ds(NS * ROWS_PER_TILE, ROWS_TAIL)])


@jax.jit
def _sc_segsum(x, src, dst, zeros):
    mesh = plsc.VectorSubcoreMesh(core_axis_name="c", subcore_axis_name="s")
    return pl.kernel(
        _segsum_body,
        out_type=jax.ShapeDtypeStruct((NC, N_NODES, DIM), jnp.float32),
        mesh=mesh,
        scratch_types=[
            pltpu.VMEM((HALF, EB), jnp.int32),
            pltpu.VMEM((HALF, EB), jnp.int32),
            pltpu.VMEM((NBUF, EB, DIM), jnp.float32),
            pltpu.VMEM_SHARED((AGG_ROWS, DIM), jnp.float32),
            [pltpu.SemaphoreType.DMA] * NBUF,
        ],
    )(x, src, dst, zeros)


ROW_BLK = 1000


def _mlp_body(h_ref, agg_ref, wa_ref, wb_ref, wc_ref, l_ref,
              hout_ref, head_ref, acc_ref):
    i = pl.program_id(0)
    z = h_ref[...] + agg_ref[0] + agg_ref[1]
    z = jnp.maximum(jnp.dot(z, wa_ref[...], preferred_element_type=jnp.float32), 0.0)
    z = jnp.maximum(jnp.dot(z, wb_ref[...], preferred_element_type=jnp.float32), 0.0)
    z = jnp.maximum(jnp.dot(z, wc_ref[...], preferred_element_type=jnp.float32), 0.0)
    hout_ref[...] = z

    @pl.when(i == 0)
    def _():
        acc_ref[...] = jnp.zeros_like(acc_ref)

    acc_ref[...] += jnp.sum(z, axis=0, keepdims=True)

    @pl.when(i == pl.num_programs(0) - 1)
    def _():
        head_ref[...] = jnp.dot(acc_ref[...] / N_NODES, l_ref[...],
                                preferred_element_type=jnp.float32)


def _final_body(h_ref, agg_ref, wa_ref, wb_ref, wc_ref, l_ref, prev_ref,
                out_ref, acc_ref):
    i = pl.program_id(0)
    z = h_ref[...] + agg_ref[0] + agg_ref[1]
    z = jnp.maximum(jnp.dot(z, wa_ref[...], preferred_element_type=jnp.float32), 0.0)
    z = jnp.maximum(jnp.dot(z, wb_ref[...], preferred_element_type=jnp.float32), 0.0)
    z = jnp.maximum(jnp.dot(z, wc_ref[...], preferred_element_type=jnp.float32), 0.0)

    @pl.when(i == 0)
    def _():
        acc_ref[...] = jnp.zeros_like(acc_ref)

    acc_ref[...] += jnp.sum(z, axis=0, keepdims=True)

    @pl.when(i == pl.num_programs(0) - 1)
    def _():
        head = jnp.dot(acc_ref[...] / N_NODES, l_ref[...],
                       preferred_element_type=jnp.float32)
        total = head + jnp.sum(prev_ref[...], axis=0, keepdims=True)
        out_ref[...] = jax.nn.sigmoid(total)


def _tc_mlp(h, agg, wa, wb, wc, l):
    grid = N_NODES // ROW_BLK
    return pl.pallas_call(
        _mlp_body,
        grid=(grid,),
        in_specs=[
            pl.BlockSpec((ROW_BLK, DIM), lambda i: (i, 0)),
            pl.BlockSpec((NC, ROW_BLK, DIM), lambda i: (0, i, 0)),
            pl.BlockSpec((DIM, DIM), lambda i: (0, 0)),
            pl.BlockSpec((DIM, DIM), lambda i: (0, 0)),
            pl.BlockSpec((DIM, DIM), lambda i: (0, 0)),
            pl.BlockSpec((DIM, 1), lambda i: (0, 0)),
        ],
        out_specs=[
            pl.BlockSpec((ROW_BLK, DIM), lambda i: (i, 0)),
            pl.BlockSpec((1, 1), lambda i: (0, 0)),
        ],
        out_shape=[
            jax.ShapeDtypeStruct((N_NODES, DIM), jnp.float32),
            jax.ShapeDtypeStruct((1, 1), jnp.float32),
        ],
        scratch_shapes=[pltpu.VMEM((1, DIM), jnp.float32)],
    )(h, agg, wa, wb, wc, l)


def _tc_final(h, agg, wa, wb, wc, l, prev):
    grid = N_NODES // ROW_BLK
    return pl.pallas_call(
        _final_body,
        grid=(grid,),
        in_specs=[
            pl.BlockSpec((ROW_BLK, DIM), lambda i: (i, 0)),
            pl.BlockSpec((NC, ROW_BLK, DIM), lambda i: (0, i, 0)),
            pl.BlockSpec((DIM, DIM), lambda i: (0, 0)),
            pl.BlockSpec((DIM, DIM), lambda i: (0, 0)),
            pl.BlockSpec((DIM, DIM), lambda i: (0, 0)),
            pl.BlockSpec((DIM, 1), lambda i: (0, 0)),
            pl.BlockSpec((4, 1), lambda i: (0, 0)),
        ],
        out_specs=pl.BlockSpec((1, 1), lambda i: (0, 0)),
        out_shape=jax.ShapeDtypeStruct((1, 1), jnp.float32),
        scratch_shapes=[pltpu.VMEM((1, DIM), jnp.float32)],
    )(h, agg, wa, wb, wc, l, prev)


def kernel(x, edge_index, Wa, Wb, Wc, L):
    src = edge_index[0].reshape(NBLK, EB)
    dst = edge_index[1].reshape(NBLK, EB)
    zeros = jnp.zeros((AGG_ROWS, DIM), jnp.float32)

    h = x
    heads = []
    for i in range(4):
        agg = _sc_segsum(h, src, dst, zeros)
        h, head = _tc_mlp(h, agg, Wa[i], Wb[i], Wc[i], L[i])
        heads.append(head)

    agg = _sc_segsum(h, src, dst, zeros)
    prev = jnp.concatenate(heads, axis=0)
    out = _tc_final(h, agg, Wa[4], Wb[4], Wc[4], L[4], prev)
    return out.reshape((1,))


# speedup vs baseline: 3.5996x; 1.0065x over previous
import functools

import jax
import jax.numpy as jnp
from jax import lax
from jax.experimental import pallas as pl
from jax.experimental.pallas import tpu as pltpu
from jax.experimental.pallas import tpu_sc as plsc

N_NODES = 10000
DIM = 128
N_EDGES = 320000

NC = 2
NS = 16
NW = NC * NS

EB = 50
BLK_PER_TILE = 200
NBLK = NW * BLK_PER_TILE
AGG_ROWS = N_NODES
ROWS_PER_TILE = 624
ROWS_TAIL = N_NODES - NS * ROWS_PER_TILE
NBUF = 5
NPHASE = 5
PHB = BLK_PER_TILE // NPHASE


def _segsum_body(x_hbm, src_hbm, dst_hbm, zeros_hbm, out_hbm,
                 idx_s, idx_d, rows, agg_sh, sems):
    c = lax.axis_index("c")
    s = lax.axis_index("s")
    wid = c * NS + s
    iboff = pl.multiple_of(wid * BLK_PER_TILE, 8)

    base = pl.multiple_of(s * ROWS_PER_TILE, 8)
    pltpu.sync_copy(zeros_hbm.at[pl.ds(base, ROWS_PER_TILE)],
                    agg_sh.at[pl.ds(base, ROWS_PER_TILE)])

    @pl.when(s == NS - 1)
    def _():
        pltpu.sync_copy(zeros_hbm.at[pl.ds(NS * ROWS_PER_TILE, ROWS_TAIL)],
                        agg_sh.at[pl.ds(NS * ROWS_PER_TILE, ROWS_TAIL)])

    plsc.subcore_barrier()

    def gather(b, blk):
        return pltpu.async_copy(x_hbm.at[idx_s.at[blk]], rows.at[b], sems[b])

    for p in range(NPHASE):
        pltpu.sync_copy(src_hbm.at[pl.ds(iboff + p * PHB, PHB)], idx_s)
        pltpu.sync_copy(dst_hbm.at[pl.ds(iboff + p * PHB, PHB)], idx_d)

        for b in range(NBUF):
            gather(b, b)

        def body(j, carry):
            for b in range(NBUF):
                blk = j + b
                pltpu.make_async_copy(x_hbm.at[idx_s.at[blk]], rows.at[b],
                                      sems[b]).wait()
                pltpu.sync_copy(rows.at[b], agg_sh.at[idx_d.at[blk]],
                                add=True)

                @pl.when(blk + NBUF < PHB)
                def _():
                    gather(b, blk + NBUF)

            return carry

        lax.fori_loop(0, PHB // NBUF, lambda j, cr: body(j * NBUF, cr), 0)

    plsc.subcore_barrier()
    pltpu.sync_copy(agg_sh.at[pl.ds(base, ROWS_PER_TILE)],
                    out_hbm.at[c, pl.ds(base, ROWS_PER_TILE)])

    @pl.when(s == NS - 1)
    def _():
        pltpu.sync_copy(agg_sh.at[pl.ds(NS * ROWS_PER_TILE, ROWS_TAIL)],
                        out_hbm.at[c, pl.ds(NS * ROWS_PER_TILE, ROWS_TAIL)])


@jax.jit
def _sc_segsum(x, src, dst, zeros):
    mesh = plsc.VectorSubcoreMesh(core_axis_name="c", subcore_axis_name="s")
    return pl.kernel(
        _segsum_body,
        out_type=jax.ShapeDtypeStruct((NC, N_NODES, DIM), jnp.float32),
        mesh=mesh,
        scratch_types=[
            pltpu.VMEM((PHB, EB), jnp.int32),
            pltpu.VMEM((PHB, EB), jnp.int32),
            pltpu.VMEM((NBUF, EB, DIM), jnp.float32),
            pltpu.VMEM_SHARED((AGG_ROWS, DIM), jnp.float32),
            [pltpu.SemaphoreType.DMA] * NBUF,
        ],
    )(x, src, dst, zeros)


ROW_BLK = 1000


def _mlp_body(h_ref, agg_ref, wa_ref, wb_ref, wc_ref, l_ref,
              hout_ref, head_ref, acc_ref):
    i = pl.program_id(0)
    z = h_ref[...] + agg_ref[0] + agg_ref[1]
    z = jnp.maximum(jnp.dot(z, wa_ref[...], preferred_element_type=jnp.float32), 0.0)
    z = jnp.maximum(jnp.dot(z, wb_ref[...], preferred_element_type=jnp.float32), 0.0)
    z = jnp.maximum(jnp.dot(z, wc_ref[...], preferred_element_type=jnp.float32), 0.0)
    hout_ref[...] = z

    @pl.when(i == 0)
    def _():
        acc_ref[...] = jnp.zeros_like(acc_ref)

    acc_ref[...] += jnp.sum(z, axis=0, keepdims=True)

    @pl.when(i == pl.num_programs(0) - 1)
    def _():
        head_ref[...] = jnp.dot(acc_ref[...] / N_NODES, l_ref[...],
                                preferred_element_type=jnp.float32)


def _final_body(h_ref, agg_ref, wa_ref, wb_ref, wc_ref, l_ref, prev_ref,
                out_ref, acc_ref):
    i = pl.program_id(0)
    z = h_ref[...] + agg_ref[0] + agg_ref[1]
    z = jnp.maximum(jnp.dot(z, wa_ref[...], preferred_element_type=jnp.float32), 0.0)
    z = jnp.maximum(jnp.dot(z, wb_ref[...], preferred_element_type=jnp.float32), 0.0)
    z = jnp.maximum(jnp.dot(z, wc_ref[...], preferred_element_type=jnp.float32), 0.0)

    @pl.when(i == 0)
    def _():
        acc_ref[...] = jnp.zeros_like(acc_ref)

    acc_ref[...] += jnp.sum(z, axis=0, keepdims=True)

    @pl.when(i == pl.num_programs(0) - 1)
    def _():
        head = jnp.dot(acc_ref[...] / N_NODES, l_ref[...],
                       preferred_element_type=jnp.float32)
        total = head + jnp.sum(prev_ref[...], axis=0, keepdims=True)
        out_ref[...] = jax.nn.sigmoid(total)


def _tc_mlp(h, agg, wa, wb, wc, l):
    grid = N_NODES // ROW_BLK
    return pl.pallas_call(
        _mlp_body,
        grid=(grid,),
        in_specs=[
            pl.BlockSpec((ROW_BLK, DIM), lambda i: (i, 0)),
            pl.BlockSpec((NC, ROW_BLK, DIM), lambda i: (0, i, 0)),
            pl.BlockSpec((DIM, DIM), lambda i: (0, 0)),
            pl.BlockSpec((DIM, DIM), lambda i: (0, 0)),
            pl.BlockSpec((DIM, DIM), lambda i: (0, 0)),
            pl.BlockSpec((DIM, 1), lambda i: (0, 0)),
        ],
        out_specs=[
            pl.BlockSpec((ROW_BLK, DIM), lambda i: (i, 0)),
            pl.BlockSpec((1, 1), lambda i: (0, 0)),
        ],
        out_shape=[
            jax.ShapeDtypeStruct((N_NODES, DIM), jnp.float32),
            jax.ShapeDtypeStruct((1, 1), jnp.float32),
        ],
        scratch_shapes=[pltpu.VMEM((1, DIM), jnp.float32)],
    )(h, agg, wa, wb, wc, l)


def _tc_final(h, agg, wa, wb, wc, l, prev):
    grid = N_NODES // ROW_BLK
    return pl.pallas_call(
        _final_body,
        grid=(grid,),
        in_specs=[
            pl.BlockSpec((ROW_BLK, DIM), lambda i: (i, 0)),
            pl.BlockSpec((NC, ROW_BLK, DIM), lambda i: (0, i, 0)),
            pl.BlockSpec((DIM, DIM), lambda i: (0, 0)),
            pl.BlockSpec((DIM, DIM), lambda i: (0, 0)),
            pl.BlockSpec((DIM, DIM), lambda i: (0, 0)),
            pl.BlockSpec((DIM, 1), lambda i: (0, 0)),
            pl.BlockSpec((4, 1), lambda i: (0, 0)),
        ],
        out_specs=pl.BlockSpec((1, 1), lambda i: (0, 0)),
        out_shape=jax.ShapeDtypeStruct((1, 1), jnp.float32),
        scratch_shapes=[pltpu.VMEM((1, DIM), jnp.float32)],
    )(h, agg, wa, wb, wc, l, prev)


def kernel(x, edge_index, Wa, Wb, Wc, L):
    src = edge_index[0].reshape(NBLK, EB)
    dst = edge_index[1].reshape(NBLK, EB)
    zeros = jnp.zeros((AGG_ROWS, DIM), jnp.float32)

    h = x
    heads = []
    for i in range(4):
        agg = _sc_segsum(h, src, dst, zeros)
        h, head = _tc_mlp(h, agg, Wa[i], Wb[i], Wc[i], L[i])
        heads.append(head)

    agg = _sc_segsum(h, src, dst, zeros)
    prev = jnp.concatenate(heads, axis=0)
    out = _tc_final(h, agg, Wa[4], Wb[4], Wc[4], L[4], prev)
    return out.reshape((1,))
